# Initial kernel scaffold; baseline (speedup 1.0000x reference)
import jax
import jax.numpy as jnp
from jax.experimental import pallas as pl


def kernel(x, edge_index, W1, b1, W2, b2, Wr, br):
    raise NotImplementedError("write your pallas kernel here")


# speedup vs baseline: 13.2308x; 13.2308x over previous
import functools

import jax
import jax.numpy as jnp
from jax import lax
from jax.experimental import pallas as pl
from jax.experimental.pallas import tpu as pltpu
from jax.experimental.pallas import tpu_sc as plsc

_NC = 2
_NS = 16
_NW = _NC * _NS
_ROW = 128


def _sc_scatter_add(p, src2d, dst2d, npad, gather_rows=True):
    N, D = p.shape
    rpw = src2d.shape[0] // _NW
    rpt = ((N // _NS + 7) // 8) * 8
    last = N - rpt * (_NS - 1)
    assert last > 0

    mesh = plsc.VectorSubcoreMesh(core_axis_name="c", subcore_axis_name="s")

    @functools.partial(
        pl.kernel,
        mesh=mesh,
        compiler_params=pltpu.CompilerParams(use_tc_tiling_on_sc=False),
        out_type=jax.ShapeDtypeStruct((_NC * N, D), jnp.float32),
        scratch_types=[
            pltpu.VMEM((rpw, _ROW), jnp.int32),
            pltpu.VMEM((rpw, _ROW), jnp.int32),
            pltpu.VMEM((_ROW, D), jnp.float32),
            pltpu.VMEM_SHARED((npad, D), jnp.float32),
            pltpu.SemaphoreType.DMA,
        ],
    )
    def k(p_hbm, src_hbm, dst_hbm, out_hbm, src_v, dst_v, rows_v, acc, sem):
        cid = lax.axis_index("c")
        sid = lax.axis_index("s")
        wid = sid * _NC + cid
        r0 = pl.multiple_of(sid * rpt, 8)
        @pl.when(sid < _NS - 1)
        def _():
            pltpu.sync_copy(p_hbm.at[pl.ds(r0, rpt)], acc.at[pl.ds(r0, rpt)])

        @pl.when(sid == _NS - 1)
        def _():
            pltpu.sync_copy(p_hbm.at[pl.ds((_NS - 1) * rpt, last)],
                            acc.at[pl.ds((_NS - 1) * rpt, last)])
        pltpu.sync_copy(src_hbm.at[pl.ds(wid * rpw, rpw)], src_v)
        pltpu.sync_copy(dst_hbm.at[pl.ds(wid * rpw, rpw)], dst_v)
        if not gather_rows:
            pltpu.sync_copy(p_hbm.at[pl.ds(0, _ROW)], rows_v)
        plsc.subcore_barrier()

        def body(j, carry):
            if gather_rows:
                pltpu.async_copy(p_hbm.at[src_v.at[j]], rows_v, sem).wait()
            pltpu.sync_copy(rows_v, acc.at[dst_v.at[j]], add=True)
            return carry

        lax.fori_loop(0, rpw, body, 0)
        plsc.subcore_barrier()
        o0 = pl.multiple_of(cid * N + r0, 8)

        @pl.when(sid < _NS - 1)
        def _():
            pltpu.sync_copy(acc.at[pl.ds(r0, rpt)], out_hbm.at[pl.ds(o0, rpt)])

        @pl.when(sid == _NS - 1)
        def _():
            pltpu.sync_copy(
                acc.at[pl.ds((_NS - 1) * rpt, last)],
                out_hbm.at[pl.ds(
                    pl.multiple_of(cid * N + (_NS - 1) * rpt, 8), last)])

    return k(p, src2d, dst2d).reshape(_NC, N, D)


def _tc_first(x, W1, c0, c1, blk=400):
    N, K = x.shape
    D = W1.shape[1]

    def body(x_ref, w_ref, c0_ref, c1_ref, o_ref):
        deg = c0_ref[:, 0:1] + c1_ref[:, 0:1] - 1.0
        dinv = lax.rsqrt(deg)
        h = jnp.dot(x_ref[...], w_ref[...], preferred_element_type=jnp.float32)
        o_ref[...] = h * dinv

    return pl.pallas_call(
        body,
        grid=(N // blk,),
        in_specs=[
            pl.BlockSpec((blk, K), lambda i: (i, 0)),
            pl.BlockSpec((K, D), lambda i: (0, 0)),
            pl.BlockSpec((blk, 16), lambda i: (i, 0)),
            pl.BlockSpec((blk, 16), lambda i: (i, 0)),
        ],
        out_specs=pl.BlockSpec((blk, D), lambda i: (i, 0)),
        out_shape=jax.ShapeDtypeStruct((N, D), jnp.float32),
    )(x, W1, c0, c1)


def _tc_mid(s0, s1, p1, c0, c1, b1, W2, blk=400):
    N, D = p1.shape
    D2 = W2.shape[1]

    def body(s0_ref, s1_ref, p_ref, c0_ref, c1_ref, b_ref, w_ref, o_ref):
        deg = c0_ref[:, 0:1] + c1_ref[:, 0:1] - 1.0
        dinv = lax.rsqrt(deg)
        s = s0_ref[...] + s1_ref[...] - p_ref[...]
        h = jnp.maximum(s * dinv + b_ref[0:1, :], 0.0)
        o_ref[...] = jnp.dot(h, w_ref[...],
                             preferred_element_type=jnp.float32) * dinv

    return pl.pallas_call(
        body,
        grid=(N // blk,),
        in_specs=[
            pl.BlockSpec((blk, D), lambda i: (i, 0)),
            pl.BlockSpec((blk, D), lambda i: (i, 0)),
            pl.BlockSpec((blk, D), lambda i: (i, 0)),
            pl.BlockSpec((blk, 16), lambda i: (i, 0)),
            pl.BlockSpec((blk, 16), lambda i: (i, 0)),
            pl.BlockSpec((8, D), lambda i: (0, 0)),
            pl.BlockSpec((D, D2), lambda i: (0, 0)),
        ],
        out_specs=pl.BlockSpec((blk, D2), lambda i: (i, 0)),
        out_shape=jax.ShapeDtypeStruct((N, D2), jnp.float32),
    )(s0, s1, p1, c0, c1, b1, W2)


def _tc_last(s0, s1, p2, c0, c1, b2, Wrp, brp, blk=400):
    N, D = p2.shape
    DO = Wrp.shape[1]

    def body(s0_ref, s1_ref, p_ref, c0_ref, c1_ref, b_ref, w_ref, br_ref,
             o_ref):
        deg = c0_ref[:, 0:1] + c1_ref[:, 0:1] - 1.0
        dinv = lax.rsqrt(deg)
        s = s0_ref[...] + s1_ref[...] - p_ref[...]
        h = jnp.maximum(s * dinv + b_ref[0:1, :], 0.0)
        o_ref[...] = jnp.dot(h, w_ref[...],
                             preferred_element_type=jnp.float32) + br_ref[0:1, :]

    return pl.pallas_call(
        body,
        grid=(N // blk,),
        in_specs=[
            pl.BlockSpec((blk, D), lambda i: (i, 0)),
            pl.BlockSpec((blk, D), lambda i: (i, 0)),
            pl.BlockSpec((blk, D), lambda i: (i, 0)),
            pl.BlockSpec((blk, 16), lambda i: (i, 0)),
            pl.BlockSpec((blk, 16), lambda i: (i, 0)),
            pl.BlockSpec((8, D), lambda i: (0, 0)),
            pl.BlockSpec((D, DO), lambda i: (0, 0)),
            pl.BlockSpec((8, DO), lambda i: (0, 0)),
        ],
        out_specs=pl.BlockSpec((blk, DO), lambda i: (i, 0)),
        out_shape=jax.ShapeDtypeStruct((N, DO), jnp.float32),
    )(s0, s1, p2, c0, c1, b2, Wrp, brp)


def kernel(x, edge_index, W1, b1, W2, b2, Wr, br):
    N = x.shape[0]
    E = edge_index.shape[1]
    npad = N + 8

    src = edge_index[0].astype(jnp.int32)
    dst = edge_index[1].astype(jnp.int32)
    rpw = ((-(-E // (_NW * _ROW)) + 7) // 8) * 8
    epad = _NW * _ROW * rpw
    src2d = jnp.concatenate(
        [src, jnp.zeros((epad - E,), jnp.int32)]).reshape(-1, _ROW)
    dst2d = jnp.concatenate(
        [dst, jnp.full((epad - E,), N, jnp.int32)]).reshape(-1, _ROW)

    ones = jnp.ones((N, 16), jnp.float32)
    cnt = _sc_scatter_add(ones, src2d, dst2d, npad, gather_rows=False)
    c0, c1 = cnt[0], cnt[1]

    p1 = _tc_first(x, W1, c0, c1)
    s1 = _sc_scatter_add(p1, src2d, dst2d, npad)
    p2 = _tc_mid(s1[0], s1[1], p1, c0, c1,
                 jnp.tile(b1[None, :], (8, 1)), W2)
    s2 = _sc_scatter_add(p2, src2d, dst2d, npad)

    DO = Wr.shape[1]
    Wrp = jnp.pad(Wr, ((0, 0), (0, 128 - DO)))
    brp = jnp.tile(jnp.pad(br, (0, 128 - DO))[None, :], (8, 1))
    out = _tc_last(s2[0], s2[1], p2, c0, c1,
                   jnp.tile(b2[None, :], (8, 1)), Wrp, brp)
    return out[:, :DO]


# speedup vs baseline: 15.1190x; 1.1427x over previous
import functools

import jax
import jax.numpy as jnp
from jax import lax
from jax.experimental import pallas as pl
from jax.experimental.pallas import tpu as pltpu
from jax.experimental.pallas import tpu_sc as plsc

_NC = 2
_NS = 16
_NW = _NC * _NS
_ROW = 128


def _sc_scatter_add(p, src2d, dst2d, npad, gather_rows=True):
    N, D = p.shape
    rpw = src2d.shape[0] // _NW
    rpt = ((N // _NS + 7) // 8) * 8
    last = N - rpt * (_NS - 1)
    assert last > 0

    mesh = plsc.VectorSubcoreMesh(core_axis_name="c", subcore_axis_name="s")

    nbuf = 8
    la = nbuf // 2
    ngrp = rpw // nbuf
    assert rpw % nbuf == 0 and ngrp >= 2

    @functools.partial(
        pl.kernel,
        mesh=mesh,
        compiler_params=pltpu.CompilerParams(use_tc_tiling_on_sc=False),
        out_type=jax.ShapeDtypeStruct((_NC * N, D), jnp.float32),
        scratch_types=[
            pltpu.VMEM((rpw, _ROW), jnp.int32),
            pltpu.VMEM((rpw, _ROW), jnp.int32),
        ]
        + [pltpu.VMEM((_ROW, D), jnp.float32) for _ in range(nbuf)]
        + [pltpu.SemaphoreType.DMA for _ in range(2 * nbuf)]
        + [pltpu.VMEM_SHARED((npad, D), jnp.float32)],
    )
    def k(p_hbm, src_hbm, dst_hbm, out_hbm, src_v, dst_v, *rest):
        rows = rest[:nbuf]
        gsem = rest[nbuf:2 * nbuf]
        ssem = rest[2 * nbuf:3 * nbuf]
        acc = rest[3 * nbuf]
        cid = lax.axis_index("c")
        sid = lax.axis_index("s")
        wid = sid * _NC + cid
        r0 = pl.multiple_of(sid * rpt, 8)
        @pl.when(sid < _NS - 1)
        def _():
            pltpu.sync_copy(p_hbm.at[pl.ds(r0, rpt)], acc.at[pl.ds(r0, rpt)])

        @pl.when(sid == _NS - 1)
        def _():
            pltpu.sync_copy(p_hbm.at[pl.ds((_NS - 1) * rpt, last)],
                            acc.at[pl.ds((_NS - 1) * rpt, last)])
        pltpu.sync_copy(src_hbm.at[pl.ds(wid * rpw, rpw)], src_v)
        pltpu.sync_copy(dst_hbm.at[pl.ds(wid * rpw, rpw)], dst_v)
        if not gather_rows:
            pltpu.sync_copy(p_hbm.at[pl.ds(0, _ROW)], rows[0])
        plsc.subcore_barrier()

        def gath(j, b):
            pltpu.async_copy(p_hbm.at[src_v.at[j]], rows[b], gsem[b])

        def gath_wait(j, b):
            pltpu.make_async_copy(p_hbm.at[src_v.at[j]], rows[b],
                                  gsem[b]).wait()

        def scat(j, b):
            pltpu.async_copy(rows[b], acc.at[dst_v.at[j]], ssem[b], add=True)

        def scat_wait(j, b):
            pltpu.make_async_copy(rows[b], acc.at[dst_v.at[j]],
                                  ssem[b]).wait()

        if gather_rows:
            def slot(j, b, do_sswait, do_gather):
                b2 = (b + la) % nbuf
                if do_gather:
                    if do_sswait:
                        scat_wait(j - la, b2)
                    gath(j + la, b2)
                gath_wait(j, b)
                scat(j, b)

            for b in range(la):
                gath(b, b)
            for b in range(nbuf):
                slot(b, b, do_sswait=(b >= la), do_gather=True)

            def group(jj, carry):
                for b in range(nbuf):
                    slot(jj * nbuf + b, b, True, True)
                return carry

            lax.fori_loop(1, ngrp - 1, group, 0)
            for b in range(nbuf):
                j = (ngrp - 1) * nbuf + b
                slot(j, b, do_sswait=(b < la), do_gather=(b < la))
            for b in range(nbuf):
                scat_wait(rpw - nbuf + b, b)
        else:
            for b in range(nbuf):
                pltpu.async_copy(rows[0], acc.at[dst_v.at[b]], ssem[b],
                                 add=True)

            def group(jj, carry):
                for b in range(nbuf):
                    j = jj * nbuf + b
                    pltpu.make_async_copy(rows[0], acc.at[dst_v.at[j - nbuf]],
                                          ssem[b]).wait()
                    pltpu.async_copy(rows[0], acc.at[dst_v.at[j]], ssem[b],
                                     add=True)
                return carry

            lax.fori_loop(1, ngrp, group, 0)
            for b in range(nbuf):
                pltpu.make_async_copy(rows[0],
                                      acc.at[dst_v.at[rpw - nbuf + b]],
                                      ssem[b]).wait()
        plsc.subcore_barrier()
        o0 = pl.multiple_of(cid * N + r0, 8)

        @pl.when(sid < _NS - 1)
        def _():
            pltpu.sync_copy(acc.at[pl.ds(r0, rpt)], out_hbm.at[pl.ds(o0, rpt)])

        @pl.when(sid == _NS - 1)
        def _():
            pltpu.sync_copy(
                acc.at[pl.ds((_NS - 1) * rpt, last)],
                out_hbm.at[pl.ds(
                    pl.multiple_of(cid * N + (_NS - 1) * rpt, 8), last)])

    return k(p, src2d, dst2d).reshape(_NC, N, D)


def _tc_first(x, W1, c0, c1, blk=400):
    N, K = x.shape
    D = W1.shape[1]

    def body(x_ref, w_ref, c0_ref, c1_ref, o_ref):
        deg = c0_ref[:, 0:1] + c1_ref[:, 0:1] - 1.0
        dinv = lax.rsqrt(deg)
        h = jnp.dot(x_ref[...], w_ref[...], preferred_element_type=jnp.float32)
        o_ref[...] = h * dinv

    return pl.pallas_call(
        body,
        grid=(N // blk,),
        in_specs=[
            pl.BlockSpec((blk, K), lambda i: (i, 0)),
            pl.BlockSpec((K, D), lambda i: (0, 0)),
            pl.BlockSpec((blk, 16), lambda i: (i, 0)),
            pl.BlockSpec((blk, 16), lambda i: (i, 0)),
        ],
        out_specs=pl.BlockSpec((blk, D), lambda i: (i, 0)),
        out_shape=jax.ShapeDtypeStruct((N, D), jnp.float32),
    )(x, W1, c0, c1)


def _tc_mid(s0, s1, p1, c0, c1, b1, W2, blk=400):
    N, D = p1.shape
    D2 = W2.shape[1]

    def body(s0_ref, s1_ref, p_ref, c0_ref, c1_ref, b_ref, w_ref, o_ref):
        deg = c0_ref[:, 0:1] + c1_ref[:, 0:1] - 1.0
        dinv = lax.rsqrt(deg)
        s = s0_ref[...] + s1_ref[...] - p_ref[...]
        h = jnp.maximum(s * dinv + b_ref[0:1, :], 0.0)
        o_ref[...] = jnp.dot(h, w_ref[...],
                             preferred_element_type=jnp.float32) * dinv

    return pl.pallas_call(
        body,
        grid=(N // blk,),
        in_specs=[
            pl.BlockSpec((blk, D), lambda i: (i, 0)),
            pl.BlockSpec((blk, D), lambda i: (i, 0)),
            pl.BlockSpec((blk, D), lambda i: (i, 0)),
            pl.BlockSpec((blk, 16), lambda i: (i, 0)),
            pl.BlockSpec((blk, 16), lambda i: (i, 0)),
            pl.BlockSpec((8, D), lambda i: (0, 0)),
            pl.BlockSpec((D, D2), lambda i: (0, 0)),
        ],
        out_specs=pl.BlockSpec((blk, D2), lambda i: (i, 0)),
        out_shape=jax.ShapeDtypeStruct((N, D2), jnp.float32),
    )(s0, s1, p1, c0, c1, b1, W2)


def _tc_last(s0, s1, p2, c0, c1, b2, Wrp, brp, blk=400):
    N, D = p2.shape
    DO = Wrp.shape[1]

    def body(s0_ref, s1_ref, p_ref, c0_ref, c1_ref, b_ref, w_ref, br_ref,
             o_ref):
        deg = c0_ref[:, 0:1] + c1_ref[:, 0:1] - 1.0
        dinv = lax.rsqrt(deg)
        s = s0_ref[...] + s1_ref[...] - p_ref[...]
        h = jnp.maximum(s * dinv + b_ref[0:1, :], 0.0)
        o_ref[...] = jnp.dot(h, w_ref[...],
                             preferred_element_type=jnp.float32) + br_ref[0:1, :]

    return pl.pallas_call(
        body,
        grid=(N // blk,),
        in_specs=[
            pl.BlockSpec((blk, D), lambda i: (i, 0)),
            pl.BlockSpec((blk, D), lambda i: (i, 0)),
            pl.BlockSpec((blk, D), lambda i: (i, 0)),
            pl.BlockSpec((blk, 16), lambda i: (i, 0)),
            pl.BlockSpec((blk, 16), lambda i: (i, 0)),
            pl.BlockSpec((8, D), lambda i: (0, 0)),
            pl.BlockSpec((D, DO), lambda i: (0, 0)),
            pl.BlockSpec((8, DO), lambda i: (0, 0)),
        ],
        out_specs=pl.BlockSpec((blk, DO), lambda i: (i, 0)),
        out_shape=jax.ShapeDtypeStruct((N, DO), jnp.float32),
    )(s0, s1, p2, c0, c1, b2, Wrp, brp)


def kernel(x, edge_index, W1, b1, W2, b2, Wr, br):
    N = x.shape[0]
    E = edge_index.shape[1]
    npad = N + 8

    src = edge_index[0].astype(jnp.int32)
    dst = edge_index[1].astype(jnp.int32)
    rpw = ((-(-E // (_NW * _ROW)) + 7) // 8) * 8
    epad = _NW * _ROW * rpw
    src2d = jnp.concatenate(
        [src, jnp.zeros((epad - E,), jnp.int32)]).reshape(-1, _ROW)
    dst2d = jnp.concatenate(
        [dst, jnp.full((epad - E,), N, jnp.int32)]).reshape(-1, _ROW)

    ones = jnp.ones((N, 16), jnp.float32)
    cnt = _sc_scatter_add(ones, src2d, dst2d, npad, gather_rows=False)
    c0, c1 = cnt[0], cnt[1]

    p1 = _tc_first(x, W1, c0, c1)
    s1 = _sc_scatter_add(p1, src2d, dst2d, npad)
    p2 = _tc_mid(s1[0], s1[1], p1, c0, c1,
                 jnp.tile(b1[None, :], (8, 1)), W2)
    s2 = _sc_scatter_add(p2, src2d, dst2d, npad)

    DO = Wr.shape[1]
    Wrp = jnp.pad(Wr, ((0, 0), (0, 128 - DO)))
    brp = jnp.tile(jnp.pad(br, (0, 128 - DO))[None, :], (8, 1))
    out = _tc_last(s2[0], s2[1], p2, c0, c1,
                   jnp.tile(b2[None, :], (8, 1)), Wrp, brp)
    return out[:, :DO]


# speedup vs baseline: 15.3340x; 1.0142x over previous
import functools

import jax
import jax.numpy as jnp
from jax import lax
from jax.experimental import pallas as pl
from jax.experimental.pallas import tpu as pltpu
from jax.experimental.pallas import tpu_sc as plsc

_NC = 2
_NS = 16
_NW = _NC * _NS
_ROW = 128
_SPLIT0 = 0.6


def _sc_scatter_add(p, src2d, dst2d, npad, gather_rows=True, split0=0.5):
    N, D = p.shape
    nbuf = 8
    la = nbuf // 2
    rtot = src2d.shape[0]
    rpw0 = int(round(rtot * split0 / (_NS * nbuf))) * nbuf
    rpw0 = max(0, min(rpw0, rtot // _NS))
    rpw1 = rtot // _NS - rpw0
    assert rpw1 % nbuf == 0 and rpw0 % nbuf == 0
    for r in (rpw0, rpw1):
        assert r == 0 or r // nbuf >= 2
    rpt = ((N // _NS + 7) // 8) * 8
    last = N - rpt * (_NS - 1)
    assert last > 0

    mesh = plsc.VectorSubcoreMesh(core_axis_name="c", subcore_axis_name="s")
    rpw = max(rpw0, rpw1)

    @functools.partial(
        pl.kernel,
        mesh=mesh,
        compiler_params=pltpu.CompilerParams(use_tc_tiling_on_sc=False),
        out_type=jax.ShapeDtypeStruct((_NC * N, D), jnp.float32),
        scratch_types=[
            pltpu.VMEM((rpw, _ROW), jnp.int32),
            pltpu.VMEM((rpw, _ROW), jnp.int32),
        ]
        + [pltpu.VMEM((_ROW, D), jnp.float32) for _ in range(nbuf)]
        + [pltpu.SemaphoreType.DMA for _ in range(2 * nbuf)]
        + [pltpu.VMEM_SHARED((npad, D), jnp.float32)],
    )
    def k(p_hbm, src_hbm, dst_hbm, out_hbm, src_v, dst_v, *rest):
        rows = rest[:nbuf]
        gsem = rest[nbuf:2 * nbuf]
        ssem = rest[2 * nbuf:3 * nbuf]
        acc = rest[3 * nbuf]
        cid = lax.axis_index("c")
        sid = lax.axis_index("s")
        r0 = pl.multiple_of(sid * rpt, 8)
        @pl.when(sid < _NS - 1)
        def _():
            pltpu.sync_copy(p_hbm.at[pl.ds(r0, rpt)], acc.at[pl.ds(r0, rpt)])

        @pl.when(sid == _NS - 1)
        def _():
            pltpu.sync_copy(p_hbm.at[pl.ds((_NS - 1) * rpt, last)],
                            acc.at[pl.ds((_NS - 1) * rpt, last)])
        if not gather_rows:
            pltpu.sync_copy(p_hbm.at[pl.ds(0, _ROW)], rows[0])

        def gath(j, b):
            pltpu.async_copy(p_hbm.at[src_v.at[j]], rows[b], gsem[b])

        def gath_wait(j, b):
            pltpu.make_async_copy(p_hbm.at[src_v.at[j]], rows[b],
                                  gsem[b]).wait()

        def scat(j, b):
            pltpu.async_copy(rows[b], acc.at[dst_v.at[j]], ssem[b], add=True)

        def scat_wait(j, b):
            pltpu.make_async_copy(rows[b], acc.at[dst_v.at[j]],
                                  ssem[b]).wait()

        def edge_phase(rpw_c, core_base):
            base = pl.multiple_of(core_base + sid * rpw_c, 8)
            pltpu.sync_copy(src_hbm.at[pl.ds(base, rpw_c)],
                            src_v.at[pl.ds(0, rpw_c)])
            pltpu.sync_copy(dst_hbm.at[pl.ds(base, rpw_c)],
                            dst_v.at[pl.ds(0, rpw_c)])
            ngrp = rpw_c // nbuf
            if gather_rows:
                def slot(j, b, do_sswait, do_gather):
                    b2 = (b + la) % nbuf
                    if do_gather:
                        if do_sswait:
                            scat_wait(j - la, b2)
                        gath(j + la, b2)
                    gath_wait(j, b)
                    scat(j, b)

                for b in range(la):
                    gath(b, b)
                for b in range(nbuf):
                    slot(b, b, do_sswait=(b >= la), do_gather=True)

                def group(jj, carry):
                    for b in range(nbuf):
                        slot(jj * nbuf + b, b, True, True)
                    return carry

                lax.fori_loop(1, ngrp - 1, group, 0)
                for b in range(nbuf):
                    j = (ngrp - 1) * nbuf + b
                    slot(j, b, do_sswait=(b < la), do_gather=(b < la))
                for b in range(nbuf):
                    scat_wait(rpw_c - nbuf + b, b)
            else:
                for b in range(nbuf):
                    pltpu.async_copy(rows[0], acc.at[dst_v.at[b]], ssem[b],
                                     add=True)

                def group(jj, carry):
                    for b in range(nbuf):
                        j = jj * nbuf + b
                        pltpu.make_async_copy(
                            rows[0], acc.at[dst_v.at[j - nbuf]],
                            ssem[b]).wait()
                        pltpu.async_copy(rows[0], acc.at[dst_v.at[j]],
                                         ssem[b], add=True)
                    return carry

                lax.fori_loop(1, ngrp, group, 0)
                for b in range(nbuf):
                    pltpu.make_async_copy(rows[0],
                                          acc.at[dst_v.at[rpw_c - nbuf + b]],
                                          ssem[b]).wait()

        if rpw0 == rpw1:
            edge_phase(rpw0, cid * (_NS * rpw0))
        else:
            if rpw0 > 0:
                @pl.when(cid == 0)
                def _():
                    edge_phase(rpw0, 0)
            if rpw1 > 0:
                @pl.when(cid == 1)
                def _():
                    edge_phase(rpw1, _NS * rpw0)
        plsc.subcore_barrier()
        o0 = pl.multiple_of(cid * N + r0, 8)

        @pl.when(sid < _NS - 1)
        def _():
            pltpu.sync_copy(acc.at[pl.ds(r0, rpt)], out_hbm.at[pl.ds(o0, rpt)])

        @pl.when(sid == _NS - 1)
        def _():
            pltpu.sync_copy(
                acc.at[pl.ds((_NS - 1) * rpt, last)],
                out_hbm.at[pl.ds(
                    pl.multiple_of(cid * N + (_NS - 1) * rpt, 8), last)])

    return k(p, src2d, dst2d).reshape(_NC, N, D)


def _tc_first(x, W1, c0, c1, blk=400):
    N, K = x.shape
    D = W1.shape[1]

    def body(x_ref, w_ref, c0_ref, c1_ref, o_ref):
        deg = c0_ref[:, 0:1] + c1_ref[:, 0:1] - 1.0
        dinv = lax.rsqrt(deg)
        h = jnp.dot(x_ref[...], w_ref[...], preferred_element_type=jnp.float32)
        o_ref[...] = h * dinv

    return pl.pallas_call(
        body,
        grid=(N // blk,),
        in_specs=[
            pl.BlockSpec((blk, K), lambda i: (i, 0)),
            pl.BlockSpec((K, D), lambda i: (0, 0)),
            pl.BlockSpec((blk, 16), lambda i: (i, 0)),
            pl.BlockSpec((blk, 16), lambda i: (i, 0)),
        ],
        out_specs=pl.BlockSpec((blk, D), lambda i: (i, 0)),
        out_shape=jax.ShapeDtypeStruct((N, D), jnp.float32),
    )(x, W1, c0, c1)


def _tc_mid(s0, s1, p1, c0, c1, b1, W2, blk=400):
    N, D = p1.shape
    D2 = W2.shape[1]

    def body(s0_ref, s1_ref, p_ref, c0_ref, c1_ref, b_ref, w_ref, o_ref):
        deg = c0_ref[:, 0:1] + c1_ref[:, 0:1] - 1.0
        dinv = lax.rsqrt(deg)
        s = s0_ref[...] + s1_ref[...] - p_ref[...]
        h = jnp.maximum(s * dinv + b_ref[0:1, :], 0.0)
        o_ref[...] = jnp.dot(h, w_ref[...],
                             preferred_element_type=jnp.float32) * dinv

    return pl.pallas_call(
        body,
        grid=(N // blk,),
        in_specs=[
            pl.BlockSpec((blk, D), lambda i: (i, 0)),
            pl.BlockSpec((blk, D), lambda i: (i, 0)),
            pl.BlockSpec((blk, D), lambda i: (i, 0)),
            pl.BlockSpec((blk, 16), lambda i: (i, 0)),
            pl.BlockSpec((blk, 16), lambda i: (i, 0)),
            pl.BlockSpec((8, D), lambda i: (0, 0)),
            pl.BlockSpec((D, D2), lambda i: (0, 0)),
        ],
        out_specs=pl.BlockSpec((blk, D2), lambda i: (i, 0)),
        out_shape=jax.ShapeDtypeStruct((N, D2), jnp.float32),
    )(s0, s1, p1, c0, c1, b1, W2)


def _tc_last(s0, s1, p2, c0, c1, b2, Wrp, brp, blk=400):
    N, D = p2.shape
    DO = Wrp.shape[1]

    def body(s0_ref, s1_ref, p_ref, c0_ref, c1_ref, b_ref, w_ref, br_ref,
             o_ref):
        deg = c0_ref[:, 0:1] + c1_ref[:, 0:1] - 1.0
        dinv = lax.rsqrt(deg)
        s = s0_ref[...] + s1_ref[...] - p_ref[...]
        h = jnp.maximum(s * dinv + b_ref[0:1, :], 0.0)
        o_ref[...] = jnp.dot(h, w_ref[...],
                             preferred_element_type=jnp.float32) + br_ref[0:1, :]

    return pl.pallas_call(
        body,
        grid=(N // blk,),
        in_specs=[
            pl.BlockSpec((blk, D), lambda i: (i, 0)),
            pl.BlockSpec((blk, D), lambda i: (i, 0)),
            pl.BlockSpec((blk, D), lambda i: (i, 0)),
            pl.BlockSpec((blk, 16), lambda i: (i, 0)),
            pl.BlockSpec((blk, 16), lambda i: (i, 0)),
            pl.BlockSpec((8, D), lambda i: (0, 0)),
            pl.BlockSpec((D, DO), lambda i: (0, 0)),
            pl.BlockSpec((8, DO), lambda i: (0, 0)),
        ],
        out_specs=pl.BlockSpec((blk, DO), lambda i: (i, 0)),
        out_shape=jax.ShapeDtypeStruct((N, DO), jnp.float32),
    )(s0, s1, p2, c0, c1, b2, Wrp, brp)


def kernel(x, edge_index, W1, b1, W2, b2, Wr, br):
    N = x.shape[0]
    E = edge_index.shape[1]
    npad = N + 8

    src = edge_index[0].astype(jnp.int32)
    dst = edge_index[1].astype(jnp.int32)
    rpw = ((-(-E // (_NW * _ROW)) + 7) // 8) * 8
    epad = _NW * _ROW * rpw
    src2d = jnp.concatenate(
        [src, jnp.zeros((epad - E,), jnp.int32)]).reshape(-1, _ROW)
    dst2d = jnp.concatenate(
        [dst, jnp.full((epad - E,), N, jnp.int32)]).reshape(-1, _ROW)

    ones = jnp.ones((N, 16), jnp.float32)
    cnt = _sc_scatter_add(ones, src2d, dst2d, npad, gather_rows=False)
    c0, c1 = cnt[0], cnt[1]

    p1 = _tc_first(x, W1, c0, c1)
    s1 = _sc_scatter_add(p1, src2d, dst2d, npad, split0=_SPLIT0)
    p2 = _tc_mid(s1[0], s1[1], p1, c0, c1,
                 jnp.tile(b1[None, :], (8, 1)), W2)
    s2 = _sc_scatter_add(p2, src2d, dst2d, npad, split0=_SPLIT0)

    DO = Wr.shape[1]
    Wrp = jnp.pad(Wr, ((0, 0), (0, 128 - DO)))
    brp = jnp.tile(jnp.pad(br, (0, 128 - DO))[None, :], (8, 1))
    out = _tc_last(s2[0], s2[1], p2, c0, c1,
                   jnp.tile(b2[None, :], (8, 1)), Wrp, brp)
    return out[:, :DO]
